# baseline (speedup 1.0000x reference)
import functools

import jax
import jax.numpy as jnp
from jax import lax
from jax.experimental import pallas as pl
from jax.experimental.pallas import tpu as pltpu
from jax.experimental.pallas import tpu_sc as plsc

N = 10000
NP = 10240
IN_DIM = 64
HID = 64
D = 128
OUT = 64
B = 8
E = 160000
NUM_MAT = 5

NC = 2
NS = 16
LANES = 16

EPT = 10112
NBLK = EPT // 128
RPT = NP // NS
B_PER_SC = B // NC

ROW_BLK = 2048


EB = 32
DEPTH = 4
NQ = EPT // (DEPTH * EB)


def _diffusion_body(x0, c0h, r0h, v0h, c1h, r1h, v1h, zeros,
                    p1, p2, p3, p4,
                    gidx_v,
                    g0, g1, g2, g3, s0, s1, s2, s3,
                    r0, r1, r2, r3, v0, v1, v2, v3, sr0, sr1, sr2, sr3,
                    acc_sh,
                    semg0, semg1, semg2, semg3, sems0, sems1, sems2, sems3):
    G = (g0, g1, g2, g3)
    S = (s0, s1, s2, s3)
    R = (r0, r1, r2, r3)
    V = (v0, v1, v2, v3)
    SR = (sr0, sr1, sr2, sr3)
    SEMG = (semg0, semg1, semg2, semg3)
    SEMS = (sems0, sems1, sems2, sems3)
    c = lax.axis_index("c")
    s = lax.axis_index("s")
    ebase = s * EPT

    def _run_stage(cols, rows, vals, src, out):
        pltpu.sync_copy(cols.at[s], gidx_v)
        bias = c * B_PER_SC * NP - NP

        @plsc.parallel_loop(0, NBLK, 1, unroll=2)
        def _(i):
            for j in range(D // LANES):
                sl = pl.ds(j * LANES, LANES)
                gidx_v[i, sl] = gidx_v[i, sl] + bias

        def _gref(blk):
            return gidx_v.at[blk // DEPTH, pl.ds((blk % DEPTH) * EB, EB)]

        def _issue_gather(blk, gb, rb, vb, sem):
            pltpu.async_copy(src.at[_gref(blk)], gb, sem)
            pltpu.async_copy(rows.at[pl.ds(ebase + blk * EB, EB)], rb, sem)
            pltpu.async_copy(vals.at[pl.ds(ebase + blk * EB, EB)], vb, sem)

        def _drain_gather(blk, gb, rb, vb, sem):
            pltpu.make_async_copy(src.at[_gref(blk)], gb, sem).wait()
            pltpu.make_async_copy(rows.at[pl.ds(ebase + blk * EB, EB)], rb, sem).wait()
            pltpu.make_async_copy(vals.at[pl.ds(ebase + blk * EB, EB)], vb, sem).wait()

        def _drain_scatter(sb, sem):
            pltpu.make_async_copy(zeros.at[pl.ds(0, EB)], sb, sem).wait()

        def _scale(gb, sb, rb, srb, vb):
            @plsc.parallel_loop(0, EB, 1, unroll=8)
            def _(e):
                v16 = plsc.load_gather(vb, [jnp.full((LANES,), e, jnp.int32)])
                loads = [gb[e, pl.ds(j * LANES, LANES)] for j in range(D // LANES)]
                for j in range(D // LANES):
                    sb[e, pl.ds(j * LANES, LANES)] = loads[j] * v16

            for j in range(EB // LANES):
                sl = pl.ds(j * LANES, LANES)
                srb[sl] = rb[sl]

        def _step(i, p):
            blk = DEPTH * i + p
            _drain_gather(blk, G[p], R[p], V[p], SEMG[p])

            @pl.when(i > 0)
            def _():
                _drain_scatter(S[p], SEMS[p])

            _scale(G[p], S[p], R[p], SR[p], V[p])
            pltpu.async_copy(S[p], acc_sh.at[SR[p]], SEMS[p], add=True)

            @pl.when(i < NQ - 1)
            def _():
                _issue_gather(blk + DEPTH, G[p], R[p], V[p], SEMG[p])

        def _b_body(b_i, carry):
            base = (c * B_PER_SC + b_i) * NP

            plsc.subcore_barrier()
            pltpu.sync_copy(zeros, acc_sh.at[pl.ds(s * RPT, RPT)])

            @plsc.parallel_loop(0, NBLK, 1, unroll=2)
            def _(i):
                for j in range(D // LANES):
                    sl = pl.ds(j * LANES, LANES)
                    gidx_v[i, sl] = gidx_v[i, sl] + NP

            plsc.subcore_barrier()

            for p in range(DEPTH):
                _issue_gather(p, G[p], R[p], V[p], SEMG[p])

            def _quad(i, carry2):
                for p in range(DEPTH):
                    _step(i, p)
                return carry2

            lax.fori_loop(0, NQ, _quad, 0)
            for p in range(DEPTH):
                _drain_scatter(S[p], SEMS[p])

            plsc.subcore_barrier()
            pltpu.sync_copy(acc_sh.at[pl.ds(s * RPT, RPT)],
                            out.at[pl.ds(base + s * RPT, RPT)])
            return carry

        lax.fori_loop(0, B_PER_SC, _b_body, 0)

    _run_stage(c0h, r0h, v0h, x0, p1)
    _run_stage(c0h, r0h, v0h, p1, p2)
    _run_stage(c1h, r1h, v1h, p1, p3)
    _run_stage(c1h, r1h, v1h, p3, p4)


def _diffusion(x0, c0, r0, v0, c1, r1, v1, zeros):
    mesh = plsc.VectorSubcoreMesh(core_axis_name="c", subcore_axis_name="s")
    sds = jax.ShapeDtypeStruct((B * NP, D), jnp.float32)
    return pl.kernel(
        _diffusion_body,
        out_type=(sds, sds, sds, sds),
        mesh=mesh,
        compiler_params=pltpu.CompilerParams(needs_layout_passes=False),
        scratch_types=(
            [pltpu.VMEM((NBLK, 128), jnp.int32)]
            + [pltpu.VMEM((EB, D), jnp.float32) for _ in range(8)]
            + [pltpu.VMEM((EB,), jnp.int32) for _ in range(4)]
            + [pltpu.VMEM((EB,), jnp.float32) for _ in range(4)]
            + [pltpu.VMEM((EB,), jnp.int32) for _ in range(4)]
            + [pltpu.VMEM_SHARED((NP, D), jnp.float32)]
            + [pltpu.SemaphoreType.DMA for _ in range(8)]
        ),
    )(x0, c0, r0, v0, c1, r1, v1, zeros)


def _prep_edges(idx, vals):
    pad = NS * EPT - E
    cols = jnp.concatenate([idx[1], jnp.zeros((pad,), jnp.int32)])
    rows = jnp.concatenate([idx[0], jnp.zeros((pad,), jnp.int32)])
    v = jnp.concatenate([vals, jnp.zeros((pad,), jnp.float32)])
    return (cols.reshape(NS, NBLK, 128), rows, v)


XB = 256


def _build_x0_body(xin_ref, st_ref, o_ref):
    o_ref[:, :, 0:IN_DIM] = xin_ref[...].reshape(B, XB, IN_DIM)
    o_ref[:, :, IN_DIM:D] = st_ref[...].reshape(B, XB, HID)


def _build_x0(inputs, state):
    grid = (NP // XB,)
    return pl.pallas_call(
        _build_x0_body,
        grid=grid,
        in_specs=[
            pl.BlockSpec((B, XB * IN_DIM), lambda i: (0, i)),
            pl.BlockSpec((B, XB * HID), lambda i: (0, i)),
        ],
        out_specs=pl.BlockSpec((B, XB, D), lambda i: (0, i, 0)),
        out_shape=jax.ShapeDtypeStruct((B, NP, D), jnp.float32),
    )(inputs, state)

def _mm5_body(x0_ref, p1_ref, p2_ref, p3_ref, p4_ref, w_ref, b_ref, o_ref):
    acc = jnp.dot(x0_ref[...], w_ref[0], preferred_element_type=jnp.float32)
    acc += jnp.dot(p1_ref[...], w_ref[1], preferred_element_type=jnp.float32)
    acc += jnp.dot(p2_ref[...], w_ref[2], preferred_element_type=jnp.float32)
    acc += jnp.dot(p3_ref[...], w_ref[3], preferred_element_type=jnp.float32)
    acc += jnp.dot(p4_ref[...], w_ref[4], preferred_element_type=jnp.float32)
    o_ref[...] = acc + b_ref[...]


def _mm5(x0, p1, p2, p3, p4, w_eff, bias_eff):
    m = x0.shape[0]
    grid = (m // ROW_BLK,)
    blk = pl.BlockSpec((ROW_BLK, D), lambda i: (i, 0))
    return pl.pallas_call(
        _mm5_body,
        grid=grid,
        in_specs=[blk, blk, blk, blk, blk,
                  pl.BlockSpec((NUM_MAT, D, OUT), lambda i: (0, 0, 0)),
                  pl.BlockSpec((1, OUT), lambda i: (0, 0))],
        out_specs=pl.BlockSpec((ROW_BLK, OUT), lambda i: (i, 0)),
        out_shape=jax.ShapeDtypeStruct((m, OUT), jnp.float32),
    )(x0, p1, p2, p3, p4, w_eff, bias_eff)


def kernel(inputs, state, support0_indices, support0_values, support1_indices,
           support1_values, weight, biases, output_size):
    batch = inputs.shape[0]
    x0 = _build_x0(inputs, state).reshape(batch * NP, D)

    c0, r0, v0 = _prep_edges(support0_indices, support0_values)
    c1, r1, v1 = _prep_edges(support1_indices, support1_values)

    zeros = jnp.zeros((RPT, D), jnp.float32)
    p1, p2, p3, p4 = _diffusion(x0, c0, r0, v0, c1, r1, v1, zeros)

    w = weight.reshape(D, NUM_MAT, OUT)
    w_eff = jnp.stack([
        w[:, 0] - w[:, 2],
        w[:, 1] - w[:, 4],
        2.0 * w[:, 2],
        w[:, 3],
        2.0 * w[:, 4],
    ], axis=0)

    out_dim = weight.shape[1]
    bias_eff = (biases + (jnp.asarray(output_size, jnp.float32) - out_dim)).reshape(1, OUT)

    res = _mm5(x0, p1, p2, p3, p4, w_eff, bias_eff)
    res = res.reshape(batch, NP, out_dim)[:, :N, :]
    return res.reshape(batch, N * out_dim)


# speedup vs baseline: 1.0097x; 1.0097x over previous
import functools

import jax
import jax.numpy as jnp
from jax import lax
from jax.experimental import pallas as pl
from jax.experimental.pallas import tpu as pltpu
from jax.experimental.pallas import tpu_sc as plsc

N = 10000
NP = 10240
IN_DIM = 64
HID = 64
D = 128
OUT = 64
B = 8
E = 160000
NUM_MAT = 5

NC = 2
NS = 16
LANES = 16

EPT = 10112
NBLK = EPT // 128
RPT = NP // NS
B_PER_SC = B // NC

ROW_BLK = 2048


EB = 16
DEPTH = 8
NQ = EPT // (DEPTH * EB)


def _diffusion_body(x0, c0h, r0h, v0h, c1h, r1h, v1h, zeros,
                    p1, p2, p3, p4,
                    gidx_v, *rest):
    G = rest[0:DEPTH]
    S = rest[DEPTH:2 * DEPTH]
    R = rest[2 * DEPTH:3 * DEPTH]
    V = rest[3 * DEPTH:4 * DEPTH]
    SR = rest[4 * DEPTH:5 * DEPTH]
    acc_sh = rest[5 * DEPTH]
    SEMG = rest[5 * DEPTH + 1:6 * DEPTH + 1]
    SEMS = rest[6 * DEPTH + 1:7 * DEPTH + 1]
    c = lax.axis_index("c")
    s = lax.axis_index("s")
    ebase = s * EPT

    def _run_stage(cols, rows, vals, src, out):
        pltpu.sync_copy(cols.at[s], gidx_v)
        bias = c * B_PER_SC * NP - NP

        @plsc.parallel_loop(0, NBLK, 1, unroll=2)
        def _(i):
            for j in range(D // LANES):
                sl = pl.ds(j * LANES, LANES)
                gidx_v[i, sl] = gidx_v[i, sl] + bias

        def _gref(blk):
            return gidx_v.at[blk // DEPTH, pl.ds((blk % DEPTH) * EB, EB)]

        def _issue_gather(blk, gb, rb, vb, sem):
            pltpu.async_copy(src.at[_gref(blk)], gb, sem)
            pltpu.async_copy(rows.at[pl.ds(ebase + blk * EB, EB)], rb, sem)
            pltpu.async_copy(vals.at[pl.ds(ebase + blk * EB, EB)], vb, sem)

        def _drain_gather(blk, gb, rb, vb, sem):
            pltpu.make_async_copy(src.at[_gref(blk)], gb, sem).wait()
            pltpu.make_async_copy(rows.at[pl.ds(ebase + blk * EB, EB)], rb, sem).wait()
            pltpu.make_async_copy(vals.at[pl.ds(ebase + blk * EB, EB)], vb, sem).wait()

        def _drain_scatter(sb, sem):
            pltpu.make_async_copy(zeros.at[pl.ds(0, EB)], sb, sem).wait()

        def _scale(gb, sb, rb, srb, vb):
            @plsc.parallel_loop(0, EB, 1, unroll=4)
            def _(e):
                v16 = plsc.load_gather(vb, [jnp.full((LANES,), e, jnp.int32)])
                loads = [gb[e, pl.ds(j * LANES, LANES)] for j in range(D // LANES)]
                for j in range(D // LANES):
                    sb[e, pl.ds(j * LANES, LANES)] = loads[j] * v16

            for j in range(EB // LANES):
                sl = pl.ds(j * LANES, LANES)
                srb[sl] = rb[sl]

        def _step(i, p):
            blk = DEPTH * i + p
            _drain_gather(blk, G[p], R[p], V[p], SEMG[p])

            @pl.when(i > 0)
            def _():
                _drain_scatter(S[p], SEMS[p])

            _scale(G[p], S[p], R[p], SR[p], V[p])
            pltpu.async_copy(S[p], acc_sh.at[SR[p]], SEMS[p], add=True)

            @pl.when(i < NQ - 1)
            def _():
                _issue_gather(blk + DEPTH, G[p], R[p], V[p], SEMG[p])

        def _b_body(b_i, carry):
            base = (c * B_PER_SC + b_i) * NP

            plsc.subcore_barrier()
            pltpu.sync_copy(zeros, acc_sh.at[pl.ds(s * RPT, RPT)])

            @plsc.parallel_loop(0, NBLK, 1, unroll=2)
            def _(i):
                for j in range(D // LANES):
                    sl = pl.ds(j * LANES, LANES)
                    gidx_v[i, sl] = gidx_v[i, sl] + NP

            plsc.subcore_barrier()

            for p in range(DEPTH):
                _issue_gather(p, G[p], R[p], V[p], SEMG[p])

            def _quad(i, carry2):
                for p in range(DEPTH):
                    _step(i, p)
                return carry2

            lax.fori_loop(0, NQ, _quad, 0)
            for p in range(DEPTH):
                _drain_scatter(S[p], SEMS[p])

            plsc.subcore_barrier()
            pltpu.sync_copy(acc_sh.at[pl.ds(s * RPT, RPT)],
                            out.at[pl.ds(base + s * RPT, RPT)])
            return carry

        lax.fori_loop(0, B_PER_SC, _b_body, 0)

    _run_stage(c0h, r0h, v0h, x0, p1)
    _run_stage(c0h, r0h, v0h, p1, p2)
    _run_stage(c1h, r1h, v1h, p1, p3)
    _run_stage(c1h, r1h, v1h, p3, p4)


def _diffusion(x0, c0, r0, v0, c1, r1, v1, zeros):
    mesh = plsc.VectorSubcoreMesh(core_axis_name="c", subcore_axis_name="s")
    sds = jax.ShapeDtypeStruct((B * NP, D), jnp.float32)
    return pl.kernel(
        _diffusion_body,
        out_type=(sds, sds, sds, sds),
        mesh=mesh,
        compiler_params=pltpu.CompilerParams(needs_layout_passes=False),
        scratch_types=(
            [pltpu.VMEM((NBLK, 128), jnp.int32)]
            + [pltpu.VMEM((EB, D), jnp.float32) for _ in range(2 * DEPTH)]
            + [pltpu.VMEM((EB,), jnp.int32) for _ in range(DEPTH)]
            + [pltpu.VMEM((EB,), jnp.float32) for _ in range(DEPTH)]
            + [pltpu.VMEM((EB,), jnp.int32) for _ in range(DEPTH)]
            + [pltpu.VMEM_SHARED((NP, D), jnp.float32)]
            + [pltpu.SemaphoreType.DMA for _ in range(2 * DEPTH)]
        ),
    )(x0, c0, r0, v0, c1, r1, v1, zeros)


def _prep_edges(idx, vals):
    pad = NS * EPT - E
    cols = jnp.concatenate([idx[1], jnp.zeros((pad,), jnp.int32)])
    rows = jnp.concatenate([idx[0], jnp.zeros((pad,), jnp.int32)])
    v = jnp.concatenate([vals, jnp.zeros((pad,), jnp.float32)])
    return (cols.reshape(NS, NBLK, 128), rows, v)


XB = 256


def _build_x0_body(xin_ref, st_ref, o_ref):
    o_ref[:, :, 0:IN_DIM] = xin_ref[...].reshape(B, XB, IN_DIM)
    o_ref[:, :, IN_DIM:D] = st_ref[...].reshape(B, XB, HID)


def _build_x0(inputs, state):
    grid = (NP // XB,)
    return pl.pallas_call(
        _build_x0_body,
        grid=grid,
        in_specs=[
            pl.BlockSpec((B, XB * IN_DIM), lambda i: (0, i)),
            pl.BlockSpec((B, XB * HID), lambda i: (0, i)),
        ],
        out_specs=pl.BlockSpec((B, XB, D), lambda i: (0, i, 0)),
        out_shape=jax.ShapeDtypeStruct((B, NP, D), jnp.float32),
    )(inputs, state)

def _mm5_body(x0_ref, p1_ref, p2_ref, p3_ref, p4_ref, w_ref, b_ref, o_ref):
    acc = jnp.dot(x0_ref[...], w_ref[0], preferred_element_type=jnp.float32)
    acc += jnp.dot(p1_ref[...], w_ref[1], preferred_element_type=jnp.float32)
    acc += jnp.dot(p2_ref[...], w_ref[2], preferred_element_type=jnp.float32)
    acc += jnp.dot(p3_ref[...], w_ref[3], preferred_element_type=jnp.float32)
    acc += jnp.dot(p4_ref[...], w_ref[4], preferred_element_type=jnp.float32)
    o_ref[...] = acc + b_ref[...]


def _mm5(x0, p1, p2, p3, p4, w_eff, bias_eff):
    m = x0.shape[0]
    grid = (m // ROW_BLK,)
    blk = pl.BlockSpec((ROW_BLK, D), lambda i: (i, 0))
    return pl.pallas_call(
        _mm5_body,
        grid=grid,
        in_specs=[blk, blk, blk, blk, blk,
                  pl.BlockSpec((NUM_MAT, D, OUT), lambda i: (0, 0, 0)),
                  pl.BlockSpec((1, OUT), lambda i: (0, 0))],
        out_specs=pl.BlockSpec((ROW_BLK, OUT), lambda i: (i, 0)),
        out_shape=jax.ShapeDtypeStruct((m, OUT), jnp.float32),
    )(x0, p1, p2, p3, p4, w_eff, bias_eff)


def kernel(inputs, state, support0_indices, support0_values, support1_indices,
           support1_values, weight, biases, output_size):
    batch = inputs.shape[0]
    x0 = _build_x0(inputs, state).reshape(batch * NP, D)

    c0, r0, v0 = _prep_edges(support0_indices, support0_values)
    c1, r1, v1 = _prep_edges(support1_indices, support1_values)

    zeros = jnp.zeros((RPT, D), jnp.float32)
    p1, p2, p3, p4 = _diffusion(x0, c0, r0, v0, c1, r1, v1, zeros)

    w = weight.reshape(D, NUM_MAT, OUT)
    w_eff = jnp.stack([
        w[:, 0] - w[:, 2],
        w[:, 1] - w[:, 4],
        2.0 * w[:, 2],
        w[:, 3],
        2.0 * w[:, 4],
    ], axis=0)

    out_dim = weight.shape[1]
    bias_eff = (biases + (jnp.asarray(output_size, jnp.float32) - out_dim)).reshape(1, OUT)

    res = _mm5(x0, p1, p2, p3, p4, w_eff, bias_eff)
    res = res.reshape(batch, NP, out_dim)[:, :N, :]
    return res.reshape(batch, N * out_dim)


# speedup vs baseline: 1.0099x; 1.0002x over previous
import jax
import jax.numpy as jnp
from jax import lax
from jax.experimental import pallas as pl
from jax.experimental.pallas import tpu as pltpu
from jax.experimental.pallas import tpu_sc as plsc

N = 10000
NP = 10240
IN_DIM = 64
HID = 64
D = 128
OUT = 64
B = 8
E = 160000
NUM_MAT = 5

NC = 2
NS = 16
LANES = 16

EPT = 10112
NBLK = EPT // 128
RPT = NP // NS
B_PER_SC = B // NC

ROW_BLK = 2048


EB = 16
DEPTH = 8
NQ = EPT // (DEPTH * EB)


def _diffusion_body(x0, c0h, r0h, v0h, c1h, r1h, v1h, zeros,
                    p1, p2, p3, p4,
                    gidx_v, *rest):
    G = rest[0:DEPTH]
    S = rest[DEPTH:2 * DEPTH]
    R = rest[2 * DEPTH:3 * DEPTH]
    V = rest[3 * DEPTH:4 * DEPTH]
    SR = rest[4 * DEPTH:5 * DEPTH]
    acc_sh = rest[5 * DEPTH]
    SEMG = rest[5 * DEPTH + 1:6 * DEPTH + 1]
    SEMS = rest[6 * DEPTH + 1:7 * DEPTH + 1]
    c = lax.axis_index("c")
    s = lax.axis_index("s")
    ebase = s * EPT

    def _run_stage(cols, rows, vals, src, out):
        pltpu.sync_copy(cols.at[s], gidx_v)
        bias = c * B_PER_SC * NP - NP

        @plsc.parallel_loop(0, NBLK, 1, unroll=2)
        def _(i):
            for j in range(D // LANES):
                sl = pl.ds(j * LANES, LANES)
                gidx_v[i, sl] = gidx_v[i, sl] + bias

        def _gref(blk):
            return gidx_v.at[blk // DEPTH, pl.ds((blk % DEPTH) * EB, EB)]

        def _issue_gather(blk, gb, rb, vb, sem):
            pltpu.async_copy(src.at[_gref(blk)], gb, sem)
            pltpu.async_copy(rows.at[pl.ds(ebase + blk * EB, EB)], rb, sem)
            pltpu.async_copy(vals.at[pl.ds(ebase + blk * EB, EB)], vb, sem)

        def _drain_gather(blk, gb, rb, vb, sem):
            pltpu.make_async_copy(src.at[_gref(blk)], gb, sem).wait()
            pltpu.make_async_copy(rows.at[pl.ds(ebase + blk * EB, EB)], rb, sem).wait()
            pltpu.make_async_copy(vals.at[pl.ds(ebase + blk * EB, EB)], vb, sem).wait()

        def _drain_scatter(sb, sem):
            pltpu.make_async_copy(zeros.at[pl.ds(0, EB)], sb, sem).wait()

        def _scale(gb, sb, rb, srb, vb):
            @plsc.parallel_loop(0, EB, 1, unroll=4)
            def _(e):
                v16 = plsc.load_gather(vb, [jnp.full((LANES,), e, jnp.int32)])
                loads = [gb[e, pl.ds(j * LANES, LANES)] for j in range(D // LANES)]
                for j in range(D // LANES):
                    sb[e, pl.ds(j * LANES, LANES)] = loads[j] * v16

            for j in range(EB // LANES):
                sl = pl.ds(j * LANES, LANES)
                srb[sl] = rb[sl]

        def _step(i, p):
            blk = DEPTH * i + p
            _drain_gather(blk, G[p], R[p], V[p], SEMG[p])

            @pl.when(i > 0)
            def _():
                _drain_scatter(S[p], SEMS[p])

            _scale(G[p], S[p], R[p], SR[p], V[p])
            pltpu.async_copy(S[p], acc_sh.at[SR[p]], SEMS[p], add=True)

            @pl.when(i < NQ - 1)
            def _():
                _issue_gather(blk + DEPTH, G[p], R[p], V[p], SEMG[p])

        def _b_body(b_i, carry):
            base = (c * B_PER_SC + b_i) * NP

            plsc.subcore_barrier()
            pltpu.sync_copy(zeros, acc_sh.at[pl.ds(s * RPT, RPT)])

            @plsc.parallel_loop(0, NBLK, 1, unroll=2)
            def _(i):
                for j in range(D // LANES):
                    sl = pl.ds(j * LANES, LANES)
                    gidx_v[i, sl] = gidx_v[i, sl] + NP

            plsc.subcore_barrier()

            for p in range(DEPTH):
                _issue_gather(p, G[p], R[p], V[p], SEMG[p])

            def _quad(i, carry2):
                for p in range(DEPTH):
                    _step(i, p)
                return carry2

            lax.fori_loop(0, NQ, _quad, 0)
            for p in range(DEPTH):
                _drain_scatter(S[p], SEMS[p])

            plsc.subcore_barrier()
            pltpu.sync_copy(acc_sh.at[pl.ds(s * RPT, RPT)],
                            out.at[pl.ds(base + s * RPT, RPT)])
            return carry

        lax.fori_loop(0, B_PER_SC, _b_body, 0)

    _run_stage(c0h, r0h, v0h, x0, p1)
    _run_stage(c0h, r0h, v0h, p1, p2)
    _run_stage(c1h, r1h, v1h, p1, p3)
    _run_stage(c1h, r1h, v1h, p3, p4)


def _diffusion(x0, c0, r0, v0, c1, r1, v1, zeros):
    mesh = plsc.VectorSubcoreMesh(core_axis_name="c", subcore_axis_name="s")
    sds = jax.ShapeDtypeStruct((B * NP, D), jnp.float32)
    return pl.kernel(
        _diffusion_body,
        out_type=(sds, sds, sds, sds),
        mesh=mesh,
        compiler_params=pltpu.CompilerParams(needs_layout_passes=False),
        scratch_types=(
            [pltpu.VMEM((NBLK, 128), jnp.int32)]
            + [pltpu.VMEM((EB, D), jnp.float32) for _ in range(2 * DEPTH)]
            + [pltpu.VMEM((EB,), jnp.int32) for _ in range(DEPTH)]
            + [pltpu.VMEM((EB,), jnp.float32) for _ in range(DEPTH)]
            + [pltpu.VMEM((EB,), jnp.int32) for _ in range(DEPTH)]
            + [pltpu.VMEM_SHARED((NP, D), jnp.float32)]
            + [pltpu.SemaphoreType.DMA for _ in range(2 * DEPTH)]
        ),
    )(x0, c0, r0, v0, c1, r1, v1, zeros)


def _prep_edges(idx, vals):
    pad = NS * EPT - E
    cols = jnp.concatenate([idx[1], jnp.zeros((pad,), jnp.int32)])
    rows = jnp.concatenate([idx[0], jnp.zeros((pad,), jnp.int32)])
    v = jnp.concatenate([vals, jnp.zeros((pad,), jnp.float32)])
    return (cols.reshape(NS, NBLK, 128), rows, v)


XB = 256


def _build_x0_body(xin_ref, st_ref, o_ref):
    o_ref[:, :, 0:IN_DIM] = xin_ref[...].reshape(B, XB, IN_DIM)
    o_ref[:, :, IN_DIM:D] = st_ref[...].reshape(B, XB, HID)


def _build_x0(inputs, state):
    grid = (NP // XB,)
    return pl.pallas_call(
        _build_x0_body,
        grid=grid,
        in_specs=[
            pl.BlockSpec((B, XB * IN_DIM), lambda i: (0, i)),
            pl.BlockSpec((B, XB * HID), lambda i: (0, i)),
        ],
        out_specs=pl.BlockSpec((B, XB, D), lambda i: (0, i, 0)),
        out_shape=jax.ShapeDtypeStruct((B, NP, D), jnp.float32),
    )(inputs, state)

def _mm5_body(x0_ref, p1_ref, p2_ref, p3_ref, p4_ref, w_ref, b_ref, o_ref):
    acc = jnp.dot(x0_ref[...], w_ref[0], preferred_element_type=jnp.float32)
    acc += jnp.dot(p1_ref[...], w_ref[1], preferred_element_type=jnp.float32)
    acc += jnp.dot(p2_ref[...], w_ref[2], preferred_element_type=jnp.float32)
    acc += jnp.dot(p3_ref[...], w_ref[3], preferred_element_type=jnp.float32)
    acc += jnp.dot(p4_ref[...], w_ref[4], preferred_element_type=jnp.float32)
    o_ref[...] = acc + b_ref[...]


def _mm5(x0, p1, p2, p3, p4, w_eff, bias_eff):
    m = x0.shape[0]
    grid = (m // ROW_BLK,)
    blk = pl.BlockSpec((ROW_BLK, D), lambda i: (i, 0))
    return pl.pallas_call(
        _mm5_body,
        grid=grid,
        in_specs=[blk, blk, blk, blk, blk,
                  pl.BlockSpec((NUM_MAT, D, OUT), lambda i: (0, 0, 0)),
                  pl.BlockSpec((1, OUT), lambda i: (0, 0))],
        out_specs=pl.BlockSpec((ROW_BLK, OUT), lambda i: (i, 0)),
        out_shape=jax.ShapeDtypeStruct((m, OUT), jnp.float32),
    )(x0, p1, p2, p3, p4, w_eff, bias_eff)


def kernel(inputs, state, support0_indices, support0_values, support1_indices,
           support1_values, weight, biases, output_size):
    batch = inputs.shape[0]
    x0 = _build_x0(inputs, state).reshape(batch * NP, D)

    c0, r0, v0 = _prep_edges(support0_indices, support0_values)
    c1, r1, v1 = _prep_edges(support1_indices, support1_values)

    zeros = jnp.zeros((RPT, D), jnp.float32)
    p1, p2, p3, p4 = _diffusion(x0, c0, r0, v0, c1, r1, v1, zeros)

    w = weight.reshape(D, NUM_MAT, OUT)
    w_eff = jnp.stack([
        w[:, 0] - w[:, 2],
        w[:, 1] - w[:, 4],
        2.0 * w[:, 2],
        w[:, 3],
        2.0 * w[:, 4],
    ], axis=0)

    out_dim = weight.shape[1]
    bias_eff = (biases + (jnp.asarray(output_size, jnp.float32) - out_dim)).reshape(1, OUT)

    res = _mm5(x0, p1, p2, p3, p4, w_eff, bias_eff)
    res = res.reshape(batch, NP, out_dim)[:, :N, :]
    return res.reshape(batch, N * out_dim)
